# baseline (speedup 1.0000x reference)
import functools

import jax
import jax.numpy as jnp
from jax import lax
from jax.experimental import pallas as pl
from jax.experimental.pallas import tpu as pltpu
from jax.experimental.pallas import tpu_sc as plsc

N = 10000
D = 128
HID = 256
NCLS = 100
NP = 10112
E = 320000
CB = 128
ECH = 2560
EPAD = ECH * CB
NSC = 2
NTILE = 16
ZR = NP // NTILE
F32 = jnp.float32
HI = lax.Precision.HIGHEST

R = 256
GR = 40


def _mesh():
    return plsc.VectorSubcoreMesh(core_axis_name="c", subcore_axis_name="s")


def _make_scatter(split_edges_across_cores):
    if split_edges_across_cores:
        cpt = ECH // (NSC * NTILE)
    else:
        cpt = ECH // NTILE

    SB = 32 if cpt % 32 == 0 else 16

    @functools.partial(
        pl.kernel,
        out_type=jax.ShapeDtypeStruct((NSC, NP, D), F32),
        mesh=_mesh(),
        scratch_types=[
            pltpu.VMEM_SHARED((NP, D), F32),
            pltpu.VMEM((SB, CB), jnp.int32),
            pltpu.VMEM((SB, CB), jnp.int32),
            pltpu.VMEM((CB, D), F32),
            pltpu.VMEM((CB, D), F32),
            pltpu.SemaphoreType.DMA,
            pltpu.SemaphoreType.DMA,
            pltpu.SemaphoreType.DMA,
            pltpu.SemaphoreType.DMA,
        ],
    )
    def k(tbl_hbm, src_hbm, dst_hbm, zeros_hbm, out_hbm,
          acc, sidx, didx, rows0, rows1, semg0, semg1, sems0, sems1):
        c = lax.axis_index("c")
        s = lax.axis_index("s")
        z = pl.ds(s * ZR, ZR)
        pltpu.sync_copy(zeros_hbm.at[z], acc.at[z])
        if split_edges_across_cores:
            base = (c * NTILE + s) * cpt
        else:
            base = s * cpt
        plsc.subcore_barrier()

        @pl.loop(0, cpt // SB)
        def _outer(b):
            if split_edges_across_cores:
                pltpu.sync_copy(src_hbm.at[pl.ds(base + b * SB, SB)], sidx)
            else:
                pltpu.sync_copy(src_hbm.at[c, pl.ds(base + b * SB, SB)], sidx)
            pltpu.sync_copy(dst_hbm.at[pl.ds(base + b * SB, SB)], didx)
            pltpu.async_copy(tbl_hbm.at[sidx.at[0]], rows0, semg0)

            @pl.loop(0, SB // 2)
            def _pair(t):
                j0 = 2 * t

                @pl.when(t > 0)
                def _():
                    pltpu.make_async_copy(rows1, acc.at[didx.at[j0 - 1]], sems1).wait()

                pltpu.async_copy(tbl_hbm.at[sidx.at[j0 + 1]], rows1, semg1)
                pltpu.make_async_copy(tbl_hbm.at[sidx.at[j0]], rows0, semg0).wait()
                pltpu.async_copy(rows0, acc.at[didx.at[j0]], sems0, add=True)
                pltpu.make_async_copy(tbl_hbm.at[sidx.at[j0 + 1]], rows1, semg1).wait()
                pltpu.async_copy(rows1, acc.at[didx.at[j0 + 1]], sems1, add=True)

                @pl.when(j0 + 2 < SB)
                def _():
                    pltpu.make_async_copy(rows0, acc.at[didx.at[j0]], sems0).wait()
                    pltpu.async_copy(tbl_hbm.at[sidx.at[j0 + 2]], rows0, semg0)

            pltpu.make_async_copy(rows0, acc.at[didx.at[SB - 2]], sems0).wait()
            pltpu.make_async_copy(rows1, acc.at[didx.at[SB - 1]], sems1).wait()

        plsc.subcore_barrier()
        pltpu.sync_copy(acc.at[z], out_hbm.at[c, z])

    return k


def _make_count():
    cpt = ECH // (NSC * NTILE)
    SB = 16

    @functools.partial(
        pl.kernel,
        out_type=jax.ShapeDtypeStruct((NSC, NP, D), F32),
        mesh=_mesh(),
        scratch_types=[
            pltpu.VMEM_SHARED((NP, D), F32),
            pltpu.VMEM((SB, CB), jnp.int32),
            pltpu.VMEM((CB, D), F32),
            pltpu.SemaphoreType.DMA,
        ],
    )
    def k(dst_hbm, zeros_hbm, ones_hbm, out_hbm, acc, didx, rows, sems):
        c = lax.axis_index("c")
        s = lax.axis_index("s")
        z = pl.ds(s * ZR, ZR)
        pltpu.sync_copy(zeros_hbm.at[z], acc.at[z])
        pltpu.sync_copy(ones_hbm, rows)
        base = (c * NTILE + s) * cpt
        plsc.subcore_barrier()

        @pl.loop(0, cpt // SB)
        def _outer(b):
            pltpu.sync_copy(dst_hbm.at[pl.ds(base + b * SB, SB)], didx)

            @pl.loop(0, SB)
            def _go(j):
                pltpu.async_copy(rows, acc.at[didx.at[j]], sems, add=True)

            @pl.loop(0, SB)
            def _drain(j):
                pltpu.make_async_copy(rows, acc.at[didx.at[0]], sems).wait()

        plsc.subcore_barrier()
        pltpu.sync_copy(acc.at[z], out_hbm.at[c, z])

    return k


_make_count = functools.cache(_make_count)


_make_scatter = functools.cache(_make_scatter)


def _dinv_of(dp):
    deg = dp[0, :, 0] + dp[1, :, 0] + 1.0
    return lax.rsqrt(deg)


def _mm1(x, xh, xs, W1, degp):
    def body(x_r, xh_r, xs_r, w_r, dp_r, o_r):
        acc = jnp.dot(x_r[...], w_r[0:D, :], preferred_element_type=F32, precision=HI)
        acc += jnp.dot(xh_r[...], w_r[D:2 * D, :], preferred_element_type=F32, precision=HI)
        acc += jnp.dot(xs_r[...], w_r[2 * D:3 * D, :], preferred_element_type=F32, precision=HI)
        dinv = _dinv_of(dp_r)
        o_r[...] = (acc * dinv[:, None]).reshape(1, R, D)

    return pl.pallas_call(
        body,
        grid=(GR, 2),
        in_specs=[
            pl.BlockSpec((R, D), lambda i, j: (i, 0)),
            pl.BlockSpec((R, D), lambda i, j: (i, 0)),
            pl.BlockSpec((R, D), lambda i, j: (i, 0)),
            pl.BlockSpec((3 * D, D), lambda i, j: (0, j)),
            pl.BlockSpec((2, R, D), lambda i, j: (0, i, 0)),
        ],
        out_specs=pl.BlockSpec((1, R, D), lambda i, j: (j, i, 0)),
        out_shape=jax.ShapeDtypeStruct((NSC, NP, D), F32),
    )(x, xh, xs, W1, degp)


def _epi1(g1, s1, degp, b1r, W2):
    def body(g_r, s_r, dp_r, b_r, w_r, o_r):
        dinv = _dinv_of(dp_r)[:, None]
        t0 = (g_r[0] + s_r[0]) * dinv + b_r[0, 0:D][None, :]
        t1 = (g_r[1] + s_r[1]) * dinv + b_r[0, D:2 * D][None, :]
        h1 = jnp.maximum(jnp.concatenate([t0, t1], axis=1), 0.0)
        h2 = jnp.dot(h1, w_r[...], preferred_element_type=F32, precision=HI)
        o_r[...] = h2 * dinv

    return pl.pallas_call(
        body,
        grid=(GR,),
        in_specs=[
            pl.BlockSpec((2, R, D), lambda i: (0, i, 0)),
            pl.BlockSpec((2, R, D), lambda i: (0, i, 0)),
            pl.BlockSpec((2, R, D), lambda i: (0, i, 0)),
            pl.BlockSpec((1, HID), lambda i: (0, 0)),
            pl.BlockSpec((HID, D), lambda i: (0, 0)),
        ],
        out_specs=pl.BlockSpec((R, D), lambda i: (i, 0)),
        out_shape=jax.ShapeDtypeStruct((NP, D), F32),
    )(g1, s1, degp, b1r, W2)


def _epi2(g2, s2, degp, b2r, clsp):
    def body(g_r, s_r, dp_r, b_r, cl_r, o_r):
        dinv = _dinv_of(dp_r)[:, None]
        h = (g_r[...] + s_r[0] + s_r[1]) * dinv + b_r[0][None, :]
        cl = cl_r[...]
        hn = jnp.sqrt(jnp.sum(h * h, axis=1, keepdims=True))
        cn = jnp.sqrt(jnp.sum(cl * cl, axis=1))
        num = lax.dot_general(h, cl, (((1,), (1,)), ((), ())),
                              preferred_element_type=F32, precision=HI)
        o_r[...] = num / jnp.maximum(hn * cn[None, :], 1e-8)

    return pl.pallas_call(
        body,
        grid=(GR,),
        in_specs=[
            pl.BlockSpec((R, D), lambda i: (i, 0)),
            pl.BlockSpec((2, R, D), lambda i: (0, i, 0)),
            pl.BlockSpec((2, R, D), lambda i: (0, i, 0)),
            pl.BlockSpec((1, D), lambda i: (0, 0)),
            pl.BlockSpec((D, D), lambda i: (0, 0)),
        ],
        out_specs=pl.BlockSpec((R, D), lambda i: (i, 0)),
        out_shape=jax.ShapeDtypeStruct((NP, D), F32),
    )(g2, s2, degp, b2r, clsp)


def kernel(x, x_h, x_s, edge_index, cls_embeddings, W1, b1, W2, b2):
    src = edge_index[0]
    dst = edge_index[1]
    pad = EPAD - E
    srcp = jnp.concatenate([src, jnp.full((pad,), N, jnp.int32)])
    dstp = jnp.concatenate([dst, N + (jnp.arange(pad, dtype=jnp.int32) % 96)])
    src2d = srcp.reshape(ECH, CB)
    dst2d = dstp.reshape(ECH, CB)
    src_l1 = jnp.stack([src2d, src2d + NP])
    zeros128 = jnp.zeros((NP, D), F32)
    b1r = b1.reshape(1, HID)
    b2r = b2.reshape(1, D)
    clsp = jnp.zeros((D, D), F32).at[0:NCLS].set(cls_embeddings)

    ones_blk = jnp.ones((CB, D), F32)
    degp = _make_count()(dst2d, zeros128, ones_blk)
    g1 = _mm1(x, x_h, x_s, W1, degp)
    s1 = _make_scatter(False)(g1.reshape(NSC * NP, D), src_l1, dst2d, zeros128)
    g2 = _epi1(g1, s1, degp, b1r, W2)
    s2 = _make_scatter(True)(g2, src2d, dst2d, zeros128)
    out = _epi2(g2, s2, degp, b2r, clsp)
    return out[0:N, 0:NCLS]


# speedup vs baseline: 1.0454x; 1.0454x over previous
import functools

import jax
import jax.numpy as jnp
from jax import lax
from jax.experimental import pallas as pl
from jax.experimental.pallas import tpu as pltpu
from jax.experimental.pallas import tpu_sc as plsc

N = 10000
D = 128
HID = 256
NCLS = 100
NP = 10112
E = 320000
CB = 128
ECH = 2560
EPAD = ECH * CB
NSC = 2
NTILE = 16
ZR = NP // NTILE
F32 = jnp.float32
HI = lax.Precision.HIGHEST

R = 256
GR = 40


def _mesh():
    return plsc.VectorSubcoreMesh(core_axis_name="c", subcore_axis_name="s")


def _make_scatter(split_edges_across_cores):
    if split_edges_across_cores:
        cpt = ECH // (NSC * NTILE)
    else:
        cpt = ECH // NTILE

    SB = 16

    @functools.partial(
        pl.kernel,
        out_type=jax.ShapeDtypeStruct((NSC, NP, D), F32),
        mesh=_mesh(),
        scratch_types=[
            pltpu.VMEM_SHARED((NP, D), F32),
            pltpu.VMEM((SB, CB), jnp.int32),
            pltpu.VMEM((SB, CB), jnp.int32),
            pltpu.VMEM((CB, D), F32),
            pltpu.VMEM((CB, D), F32),
            pltpu.SemaphoreType.DMA,
            pltpu.SemaphoreType.DMA,
        ],
    )
    def k(tbl_hbm, src_hbm, dst_hbm, zeros_hbm, out_hbm,
          acc, sidx, didx, rows0, rows1, sem0, sem1):
        c = lax.axis_index("c")
        s = lax.axis_index("s")
        z = pl.ds(s * ZR, ZR)
        pltpu.sync_copy(zeros_hbm.at[z], acc.at[z])
        if split_edges_across_cores:
            base = (c * NTILE + s) * cpt
        else:
            base = s * cpt
        plsc.subcore_barrier()

        @pl.loop(0, cpt // SB)
        def _outer(b):
            if split_edges_across_cores:
                pltpu.sync_copy(src_hbm.at[pl.ds(base + b * SB, SB)], sidx)
            else:
                pltpu.sync_copy(src_hbm.at[c, pl.ds(base + b * SB, SB)], sidx)
            pltpu.sync_copy(dst_hbm.at[pl.ds(base + b * SB, SB)], didx)
            pltpu.async_copy(tbl_hbm.at[sidx.at[0]], rows0, sem0)

            @pl.loop(0, SB // 2)
            def _pair(t):
                j0 = 2 * t
                pltpu.async_copy(tbl_hbm.at[sidx.at[j0 + 1]], rows1, sem1)
                pltpu.make_async_copy(tbl_hbm.at[sidx.at[j0]], rows0, sem0).wait()
                pltpu.sync_copy(rows0, acc.at[didx.at[j0]], add=True)

                @pl.when(j0 + 2 < SB)
                def _():
                    pltpu.async_copy(tbl_hbm.at[sidx.at[j0 + 2]], rows0, sem0)

                pltpu.make_async_copy(tbl_hbm.at[sidx.at[j0 + 1]], rows1, sem1).wait()
                pltpu.sync_copy(rows1, acc.at[didx.at[j0 + 1]], add=True)

        plsc.subcore_barrier()
        pltpu.sync_copy(acc.at[z], out_hbm.at[c, z])

    return k


def _make_count():
    cpt = ECH // (NSC * NTILE)
    SB = 16

    @functools.partial(
        pl.kernel,
        out_type=jax.ShapeDtypeStruct((NSC, NP, D), F32),
        mesh=_mesh(),
        scratch_types=[
            pltpu.VMEM_SHARED((NP, D), F32),
            pltpu.VMEM((SB, CB), jnp.int32),
            pltpu.VMEM((CB, D), F32),
            pltpu.SemaphoreType.DMA,
        ],
    )
    def k(dst_hbm, zeros_hbm, ones_hbm, out_hbm, acc, didx, rows, sems):
        c = lax.axis_index("c")
        s = lax.axis_index("s")
        z = pl.ds(s * ZR, ZR)
        pltpu.sync_copy(zeros_hbm.at[z], acc.at[z])
        pltpu.sync_copy(ones_hbm, rows)
        base = (c * NTILE + s) * cpt
        plsc.subcore_barrier()

        @pl.loop(0, cpt // SB)
        def _outer(b):
            pltpu.sync_copy(dst_hbm.at[pl.ds(base + b * SB, SB)], didx)

            @pl.loop(0, SB)
            def _go(j):
                pltpu.async_copy(rows, acc.at[didx.at[j]], sems, add=True)

            @pl.loop(0, SB)
            def _drain(j):
                pltpu.make_async_copy(rows, acc.at[didx.at[0]], sems).wait()

        plsc.subcore_barrier()
        pltpu.sync_copy(acc.at[z], out_hbm.at[c, z])

    return k


_make_count = functools.cache(_make_count)


_make_scatter = functools.cache(_make_scatter)


def _dinv_of(dp):
    deg = dp[0, :, 0] + dp[1, :, 0] + 1.0
    return lax.rsqrt(deg)


def _mm1(x, xh, xs, W1, degp):
    def body(x_r, xh_r, xs_r, w_r, dp_r, o_r):
        acc = jnp.dot(x_r[...], w_r[0:D, :], preferred_element_type=F32, precision=HI)
        acc += jnp.dot(xh_r[...], w_r[D:2 * D, :], preferred_element_type=F32, precision=HI)
        acc += jnp.dot(xs_r[...], w_r[2 * D:3 * D, :], preferred_element_type=F32, precision=HI)
        dinv = _dinv_of(dp_r)
        o_r[...] = (acc * dinv[:, None]).reshape(1, R, D)

    return pl.pallas_call(
        body,
        grid=(GR, 2),
        in_specs=[
            pl.BlockSpec((R, D), lambda i, j: (i, 0)),
            pl.BlockSpec((R, D), lambda i, j: (i, 0)),
            pl.BlockSpec((R, D), lambda i, j: (i, 0)),
            pl.BlockSpec((3 * D, D), lambda i, j: (0, j)),
            pl.BlockSpec((2, R, D), lambda i, j: (0, i, 0)),
        ],
        out_specs=pl.BlockSpec((1, R, D), lambda i, j: (j, i, 0)),
        out_shape=jax.ShapeDtypeStruct((NSC, NP, D), F32),
    )(x, xh, xs, W1, degp)


def _epi1(g1, s1, degp, b1r, W2):
    def body(g_r, s_r, dp_r, b_r, w_r, o_r):
        dinv = _dinv_of(dp_r)[:, None]
        t0 = (g_r[0] + s_r[0]) * dinv + b_r[0, 0:D][None, :]
        t1 = (g_r[1] + s_r[1]) * dinv + b_r[0, D:2 * D][None, :]
        h1 = jnp.maximum(jnp.concatenate([t0, t1], axis=1), 0.0)
        h2 = jnp.dot(h1, w_r[...], preferred_element_type=F32, precision=HI)
        o_r[...] = h2 * dinv

    return pl.pallas_call(
        body,
        grid=(GR,),
        in_specs=[
            pl.BlockSpec((2, R, D), lambda i: (0, i, 0)),
            pl.BlockSpec((2, R, D), lambda i: (0, i, 0)),
            pl.BlockSpec((2, R, D), lambda i: (0, i, 0)),
            pl.BlockSpec((1, HID), lambda i: (0, 0)),
            pl.BlockSpec((HID, D), lambda i: (0, 0)),
        ],
        out_specs=pl.BlockSpec((R, D), lambda i: (i, 0)),
        out_shape=jax.ShapeDtypeStruct((NP, D), F32),
    )(g1, s1, degp, b1r, W2)


def _epi2(g2, s2, degp, b2r, clsp):
    def body(g_r, s_r, dp_r, b_r, cl_r, o_r):
        dinv = _dinv_of(dp_r)[:, None]
        h = (g_r[...] + s_r[0] + s_r[1]) * dinv + b_r[0][None, :]
        cl = cl_r[...]
        hn = jnp.sqrt(jnp.sum(h * h, axis=1, keepdims=True))
        cn = jnp.sqrt(jnp.sum(cl * cl, axis=1))
        num = lax.dot_general(h, cl, (((1,), (1,)), ((), ())),
                              preferred_element_type=F32, precision=HI)
        o_r[...] = num / jnp.maximum(hn * cn[None, :], 1e-8)

    return pl.pallas_call(
        body,
        grid=(GR,),
        in_specs=[
            pl.BlockSpec((R, D), lambda i: (i, 0)),
            pl.BlockSpec((2, R, D), lambda i: (0, i, 0)),
            pl.BlockSpec((2, R, D), lambda i: (0, i, 0)),
            pl.BlockSpec((1, D), lambda i: (0, 0)),
            pl.BlockSpec((D, D), lambda i: (0, 0)),
        ],
        out_specs=pl.BlockSpec((R, D), lambda i: (i, 0)),
        out_shape=jax.ShapeDtypeStruct((NP, D), F32),
    )(g2, s2, degp, b2r, clsp)


def kernel(x, x_h, x_s, edge_index, cls_embeddings, W1, b1, W2, b2):
    src = edge_index[0]
    dst = edge_index[1]
    pad = EPAD - E
    srcp = jnp.concatenate([src, jnp.full((pad,), N, jnp.int32)])
    dstp = jnp.concatenate([dst, N + (jnp.arange(pad, dtype=jnp.int32) % 96)])
    src2d = srcp.reshape(ECH, CB)
    dst2d = dstp.reshape(ECH, CB)
    src_l1 = jnp.stack([src2d, src2d + NP])
    zeros128 = jnp.zeros((NP, D), F32)
    b1r = b1.reshape(1, HID)
    b2r = b2.reshape(1, D)
    clsp = jnp.zeros((D, D), F32).at[0:NCLS].set(cls_embeddings)

    ones_blk = jnp.ones((CB, D), F32)
    degp = _make_count()(dst2d, zeros128, ones_blk)
    g1 = _mm1(x, x_h, x_s, W1, degp)
    s1 = _make_scatter(False)(g1.reshape(NSC * NP, D), src_l1, dst2d, zeros128)
    g2 = _epi1(g1, s1, degp, b1r, W2)
    s2 = _make_scatter(True)(g2, src2d, dst2d, zeros128)
    out = _epi2(g2, s2, degp, b2r, clsp)
    return out[0:N, 0:NCLS]


# speedup vs baseline: 1.0558x; 1.0099x over previous
import functools

import jax
import jax.numpy as jnp
from jax import lax
from jax.experimental import pallas as pl
from jax.experimental.pallas import tpu as pltpu
from jax.experimental.pallas import tpu_sc as plsc

N = 10000
D = 128
HID = 256
NCLS = 100
NP = 10112
E = 320000
CB = 128
ECH = 2560
EPAD = ECH * CB
NSC = 2
NTILE = 16
ZR = NP // NTILE
F32 = jnp.float32
HI = lax.Precision.HIGHEST

R = 256
GR = 40


def _mesh():
    return plsc.VectorSubcoreMesh(core_axis_name="c", subcore_axis_name="s")


def _make_scatter(split_edges_across_cores):
    if split_edges_across_cores:
        cpt = ECH // (NSC * NTILE)
    else:
        cpt = ECH // NTILE

    SB = 32 if cpt % 32 == 0 else 16

    @functools.partial(
        pl.kernel,
        out_type=jax.ShapeDtypeStruct((NSC, NP, D), F32),
        mesh=_mesh(),
        scratch_types=[
            pltpu.VMEM_SHARED((NP, D), F32),
            pltpu.VMEM((SB, CB), jnp.int32),
            pltpu.VMEM((SB, CB), jnp.int32),
            pltpu.VMEM((CB, D), F32),
            pltpu.VMEM((CB, D), F32),
            pltpu.SemaphoreType.DMA,
            pltpu.SemaphoreType.DMA,
        ],
    )
    def k(tbl_hbm, src_hbm, dst_hbm, zeros_hbm, out_hbm,
          acc, sidx, didx, rows0, rows1, sem0, sem1):
        c = lax.axis_index("c")
        s = lax.axis_index("s")
        z = pl.ds(s * ZR, ZR)
        pltpu.sync_copy(zeros_hbm.at[z], acc.at[z])
        if split_edges_across_cores:
            base = (c * NTILE + s) * cpt
        else:
            base = s * cpt
        plsc.subcore_barrier()

        @pl.loop(0, cpt // SB)
        def _outer(b):
            if split_edges_across_cores:
                pltpu.sync_copy(src_hbm.at[pl.ds(base + b * SB, SB)], sidx)
            else:
                pltpu.sync_copy(src_hbm.at[c, pl.ds(base + b * SB, SB)], sidx)
            pltpu.sync_copy(dst_hbm.at[pl.ds(base + b * SB, SB)], didx)
            pltpu.async_copy(tbl_hbm.at[sidx.at[0]], rows0, sem0)

            @pl.loop(0, SB // 2)
            def _pair(t):
                j0 = 2 * t
                pltpu.async_copy(tbl_hbm.at[sidx.at[j0 + 1]], rows1, sem1)
                pltpu.make_async_copy(tbl_hbm.at[sidx.at[j0]], rows0, sem0).wait()
                pltpu.sync_copy(rows0, acc.at[didx.at[j0]], add=True)

                @pl.when(j0 + 2 < SB)
                def _():
                    pltpu.async_copy(tbl_hbm.at[sidx.at[j0 + 2]], rows0, sem0)

                pltpu.make_async_copy(tbl_hbm.at[sidx.at[j0 + 1]], rows1, sem1).wait()
                pltpu.sync_copy(rows1, acc.at[didx.at[j0 + 1]], add=True)

        plsc.subcore_barrier()
        pltpu.sync_copy(acc.at[z], out_hbm.at[c, z])

    return k


def _make_count():
    cpt = ECH // (NSC * NTILE)
    SB = 16

    @functools.partial(
        pl.kernel,
        out_type=jax.ShapeDtypeStruct((NSC, NP, D), F32),
        mesh=_mesh(),
        scratch_types=[
            pltpu.VMEM_SHARED((NP, D), F32),
            pltpu.VMEM((SB, CB), jnp.int32),
            pltpu.VMEM((CB, D), F32),
            pltpu.SemaphoreType.DMA,
        ],
    )
    def k(dst_hbm, zeros_hbm, ones_hbm, out_hbm, acc, didx, rows, sems):
        c = lax.axis_index("c")
        s = lax.axis_index("s")
        z = pl.ds(s * ZR, ZR)
        pltpu.sync_copy(zeros_hbm.at[z], acc.at[z])
        pltpu.sync_copy(ones_hbm, rows)
        base = (c * NTILE + s) * cpt
        plsc.subcore_barrier()

        @pl.loop(0, cpt // SB)
        def _outer(b):
            pltpu.sync_copy(dst_hbm.at[pl.ds(base + b * SB, SB)], didx)

            @pl.loop(0, SB)
            def _go(j):
                pltpu.async_copy(rows, acc.at[didx.at[j]], sems, add=True)

            @pl.loop(0, SB)
            def _drain(j):
                pltpu.make_async_copy(rows, acc.at[didx.at[0]], sems).wait()

        plsc.subcore_barrier()
        pltpu.sync_copy(acc.at[z], out_hbm.at[c, z])

    return k


_make_count = functools.cache(_make_count)


_make_scatter = functools.cache(_make_scatter)


def _dinv_of(dp):
    deg = dp[0, :, 0] + dp[1, :, 0] + 1.0
    return lax.rsqrt(deg)


def _mm1(x, xh, xs, W1, degp):
    def body(x_r, xh_r, xs_r, w_r, dp_r, o_r):
        acc = jnp.dot(x_r[...], w_r[0:D, :], preferred_element_type=F32, precision=HI)
        acc += jnp.dot(xh_r[...], w_r[D:2 * D, :], preferred_element_type=F32, precision=HI)
        acc += jnp.dot(xs_r[...], w_r[2 * D:3 * D, :], preferred_element_type=F32, precision=HI)
        dinv = _dinv_of(dp_r)
        o_r[...] = (acc * dinv[:, None]).reshape(1, R, D)

    return pl.pallas_call(
        body,
        grid=(GR, 2),
        in_specs=[
            pl.BlockSpec((R, D), lambda i, j: (i, 0)),
            pl.BlockSpec((R, D), lambda i, j: (i, 0)),
            pl.BlockSpec((R, D), lambda i, j: (i, 0)),
            pl.BlockSpec((3 * D, D), lambda i, j: (0, j)),
            pl.BlockSpec((2, R, D), lambda i, j: (0, i, 0)),
        ],
        out_specs=pl.BlockSpec((1, R, D), lambda i, j: (j, i, 0)),
        out_shape=jax.ShapeDtypeStruct((NSC, NP, D), F32),
    )(x, xh, xs, W1, degp)


def _epi1(g1, s1, degp, b1r, W2):
    def body(g_r, s_r, dp_r, b_r, w_r, o_r):
        dinv = _dinv_of(dp_r)[:, None]
        t0 = (g_r[0] + s_r[0]) * dinv + b_r[0, 0:D][None, :]
        t1 = (g_r[1] + s_r[1]) * dinv + b_r[0, D:2 * D][None, :]
        h1 = jnp.maximum(jnp.concatenate([t0, t1], axis=1), 0.0)
        h2 = jnp.dot(h1, w_r[...], preferred_element_type=F32, precision=HI)
        o_r[...] = h2 * dinv

    return pl.pallas_call(
        body,
        grid=(GR,),
        in_specs=[
            pl.BlockSpec((2, R, D), lambda i: (0, i, 0)),
            pl.BlockSpec((2, R, D), lambda i: (0, i, 0)),
            pl.BlockSpec((2, R, D), lambda i: (0, i, 0)),
            pl.BlockSpec((1, HID), lambda i: (0, 0)),
            pl.BlockSpec((HID, D), lambda i: (0, 0)),
        ],
        out_specs=pl.BlockSpec((R, D), lambda i: (i, 0)),
        out_shape=jax.ShapeDtypeStruct((NP, D), F32),
    )(g1, s1, degp, b1r, W2)


def _epi2(g2, s2, degp, b2r, clsp):
    def body(g_r, s_r, dp_r, b_r, cl_r, o_r):
        dinv = _dinv_of(dp_r)[:, None]
        h = (g_r[...] + s_r[0] + s_r[1]) * dinv + b_r[0][None, :]
        cl = cl_r[...]
        hn = jnp.sqrt(jnp.sum(h * h, axis=1, keepdims=True))
        cn = jnp.sqrt(jnp.sum(cl * cl, axis=1))
        num = lax.dot_general(h, cl, (((1,), (1,)), ((), ())),
                              preferred_element_type=F32, precision=HI)
        o_r[...] = num / jnp.maximum(hn * cn[None, :], 1e-8)

    return pl.pallas_call(
        body,
        grid=(GR,),
        in_specs=[
            pl.BlockSpec((R, D), lambda i: (i, 0)),
            pl.BlockSpec((2, R, D), lambda i: (0, i, 0)),
            pl.BlockSpec((2, R, D), lambda i: (0, i, 0)),
            pl.BlockSpec((1, D), lambda i: (0, 0)),
            pl.BlockSpec((D, D), lambda i: (0, 0)),
        ],
        out_specs=pl.BlockSpec((R, D), lambda i: (i, 0)),
        out_shape=jax.ShapeDtypeStruct((NP, D), F32),
    )(g2, s2, degp, b2r, clsp)


def kernel(x, x_h, x_s, edge_index, cls_embeddings, W1, b1, W2, b2):
    src = edge_index[0]
    dst = edge_index[1]
    pad = EPAD - E
    srcp = jnp.concatenate([src, jnp.full((pad,), N, jnp.int32)])
    dstp = jnp.concatenate([dst, N + (jnp.arange(pad, dtype=jnp.int32) % 96)])
    src2d = srcp.reshape(ECH, CB)
    dst2d = dstp.reshape(ECH, CB)
    src_l1 = jnp.stack([src2d, src2d + NP])
    zeros128 = jnp.zeros((NP, D), F32)
    b1r = b1.reshape(1, HID)
    b2r = b2.reshape(1, D)
    clsp = jnp.zeros((D, D), F32).at[0:NCLS].set(cls_embeddings)

    ones_blk = jnp.ones((CB, D), F32)
    degp = _make_count()(dst2d, zeros128, ones_blk)
    g1 = _mm1(x, x_h, x_s, W1, degp)
    s1 = _make_scatter(False)(g1.reshape(NSC * NP, D), src_l1, dst2d, zeros128)
    g2 = _epi1(g1, s1, degp, b1r, W2)
    s2 = _make_scatter(True)(g2, src2d, dst2d, zeros128)
    out = _epi2(g2, s2, degp, b2r, clsp)
    return out[0:N, 0:NCLS]


# speedup vs baseline: 1.0628x; 1.0066x over previous
import functools

import jax
import jax.numpy as jnp
from jax import lax
from jax.experimental import pallas as pl
from jax.experimental.pallas import tpu as pltpu
from jax.experimental.pallas import tpu_sc as plsc

N = 10000
D = 128
HID = 256
NCLS = 100
NP = 10112
E = 320000
CB = 128
ECH = 2560
EPAD = ECH * CB
NSC = 2
NTILE = 16
ZR = NP // NTILE
F32 = jnp.float32
HI = lax.Precision.HIGHEST

R = 256
GR = 40


def _mesh():
    return plsc.VectorSubcoreMesh(core_axis_name="c", subcore_axis_name="s")


def _make_scatter(split_edges_across_cores):
    if split_edges_across_cores:
        cpt = ECH // (NSC * NTILE)
    else:
        cpt = ECH // NTILE

    SB = 32 if cpt % 32 == 0 else 40

    @functools.partial(
        pl.kernel,
        out_type=jax.ShapeDtypeStruct((NSC, NP, D), F32),
        mesh=_mesh(),
        scratch_types=[
            pltpu.VMEM_SHARED((NP, D), F32),
            pltpu.VMEM((SB, CB), jnp.int32),
            pltpu.VMEM((SB, CB), jnp.int32),
            pltpu.VMEM((CB, D), F32),
            pltpu.VMEM((CB, D), F32),
            pltpu.SemaphoreType.DMA,
            pltpu.SemaphoreType.DMA,
        ],
    )
    def k(tbl_hbm, src_hbm, dst_hbm, zeros_hbm, out_hbm,
          acc, sidx, didx, rows0, rows1, sem0, sem1):
        c = lax.axis_index("c")
        s = lax.axis_index("s")
        z = pl.ds(s * ZR, ZR)
        pltpu.sync_copy(zeros_hbm.at[z], acc.at[z])
        if split_edges_across_cores:
            base = (c * NTILE + s) * cpt
        else:
            base = s * cpt
        plsc.subcore_barrier()

        @pl.loop(0, cpt // SB)
        def _outer(b):
            if split_edges_across_cores:
                pltpu.sync_copy(src_hbm.at[pl.ds(base + b * SB, SB)], sidx)
            else:
                pltpu.sync_copy(src_hbm.at[c, pl.ds(base + b * SB, SB)], sidx)
            pltpu.sync_copy(dst_hbm.at[pl.ds(base + b * SB, SB)], didx)
            pltpu.async_copy(tbl_hbm.at[sidx.at[0]], rows0, sem0)

            @pl.loop(0, SB // 2)
            def _pair(t):
                j0 = 2 * t
                pltpu.async_copy(tbl_hbm.at[sidx.at[j0 + 1]], rows1, sem1)
                pltpu.make_async_copy(tbl_hbm.at[sidx.at[j0]], rows0, sem0).wait()
                pltpu.sync_copy(rows0, acc.at[didx.at[j0]], add=True)

                @pl.when(j0 + 2 < SB)
                def _():
                    pltpu.async_copy(tbl_hbm.at[sidx.at[j0 + 2]], rows0, sem0)

                pltpu.make_async_copy(tbl_hbm.at[sidx.at[j0 + 1]], rows1, sem1).wait()
                pltpu.sync_copy(rows1, acc.at[didx.at[j0 + 1]], add=True)

        plsc.subcore_barrier()
        pltpu.sync_copy(acc.at[z], out_hbm.at[c, z])

    return k


def _make_count():
    cpt = ECH // (NSC * NTILE)
    SB = 16

    @functools.partial(
        pl.kernel,
        out_type=jax.ShapeDtypeStruct((NSC, NP, D), F32),
        mesh=_mesh(),
        scratch_types=[
            pltpu.VMEM_SHARED((NP, D), F32),
            pltpu.VMEM((SB, CB), jnp.int32),
            pltpu.VMEM((CB, D), F32),
            pltpu.SemaphoreType.DMA,
        ],
    )
    def k(dst_hbm, zeros_hbm, ones_hbm, out_hbm, acc, didx, rows, sems):
        c = lax.axis_index("c")
        s = lax.axis_index("s")
        z = pl.ds(s * ZR, ZR)
        pltpu.sync_copy(zeros_hbm.at[z], acc.at[z])
        pltpu.sync_copy(ones_hbm, rows)
        base = (c * NTILE + s) * cpt
        plsc.subcore_barrier()

        @pl.loop(0, cpt // SB)
        def _outer(b):
            pltpu.sync_copy(dst_hbm.at[pl.ds(base + b * SB, SB)], didx)

            @pl.loop(0, SB)
            def _go(j):
                pltpu.async_copy(rows, acc.at[didx.at[j]], sems, add=True)

            @pl.loop(0, SB)
            def _drain(j):
                pltpu.make_async_copy(rows, acc.at[didx.at[0]], sems).wait()

        plsc.subcore_barrier()
        pltpu.sync_copy(acc.at[z], out_hbm.at[c, z])

    return k


_make_count = functools.cache(_make_count)


_make_scatter = functools.cache(_make_scatter)


def _dinv_of(dp):
    deg = dp[0, :, 0] + dp[1, :, 0] + 1.0
    return lax.rsqrt(deg)


def _mm1(x, xh, xs, W1, degp):
    def body(x_r, xh_r, xs_r, w_r, dp_r, o_r):
        acc = jnp.dot(x_r[...], w_r[0:D, :], preferred_element_type=F32, precision=HI)
        acc += jnp.dot(xh_r[...], w_r[D:2 * D, :], preferred_element_type=F32, precision=HI)
        acc += jnp.dot(xs_r[...], w_r[2 * D:3 * D, :], preferred_element_type=F32, precision=HI)
        dinv = _dinv_of(dp_r)
        o_r[...] = (acc * dinv[:, None]).reshape(1, R, D)

    return pl.pallas_call(
        body,
        grid=(GR, 2),
        in_specs=[
            pl.BlockSpec((R, D), lambda i, j: (i, 0)),
            pl.BlockSpec((R, D), lambda i, j: (i, 0)),
            pl.BlockSpec((R, D), lambda i, j: (i, 0)),
            pl.BlockSpec((3 * D, D), lambda i, j: (0, j)),
            pl.BlockSpec((2, R, D), lambda i, j: (0, i, 0)),
        ],
        out_specs=pl.BlockSpec((1, R, D), lambda i, j: (j, i, 0)),
        out_shape=jax.ShapeDtypeStruct((NSC, NP, D), F32),
    )(x, xh, xs, W1, degp)


def _epi1(g1, s1, degp, b1r, W2):
    def body(g_r, s_r, dp_r, b_r, w_r, o_r):
        dinv = _dinv_of(dp_r)[:, None]
        t0 = (g_r[0] + s_r[0]) * dinv + b_r[0, 0:D][None, :]
        t1 = (g_r[1] + s_r[1]) * dinv + b_r[0, D:2 * D][None, :]
        h1 = jnp.maximum(jnp.concatenate([t0, t1], axis=1), 0.0)
        h2 = jnp.dot(h1, w_r[...], preferred_element_type=F32, precision=HI)
        o_r[...] = h2 * dinv

    return pl.pallas_call(
        body,
        grid=(GR,),
        in_specs=[
            pl.BlockSpec((2, R, D), lambda i: (0, i, 0)),
            pl.BlockSpec((2, R, D), lambda i: (0, i, 0)),
            pl.BlockSpec((2, R, D), lambda i: (0, i, 0)),
            pl.BlockSpec((1, HID), lambda i: (0, 0)),
            pl.BlockSpec((HID, D), lambda i: (0, 0)),
        ],
        out_specs=pl.BlockSpec((R, D), lambda i: (i, 0)),
        out_shape=jax.ShapeDtypeStruct((NP, D), F32),
    )(g1, s1, degp, b1r, W2)


def _epi2(g2, s2, degp, b2r, clsp):
    def body(g_r, s_r, dp_r, b_r, cl_r, o_r):
        dinv = _dinv_of(dp_r)[:, None]
        h = (g_r[...] + s_r[0] + s_r[1]) * dinv + b_r[0][None, :]
        cl = cl_r[...]
        hn = jnp.sqrt(jnp.sum(h * h, axis=1, keepdims=True))
        cn = jnp.sqrt(jnp.sum(cl * cl, axis=1))
        num = lax.dot_general(h, cl, (((1,), (1,)), ((), ())),
                              preferred_element_type=F32, precision=HI)
        o_r[...] = num / jnp.maximum(hn * cn[None, :], 1e-8)

    return pl.pallas_call(
        body,
        grid=(GR,),
        in_specs=[
            pl.BlockSpec((R, D), lambda i: (i, 0)),
            pl.BlockSpec((2, R, D), lambda i: (0, i, 0)),
            pl.BlockSpec((2, R, D), lambda i: (0, i, 0)),
            pl.BlockSpec((1, D), lambda i: (0, 0)),
            pl.BlockSpec((D, D), lambda i: (0, 0)),
        ],
        out_specs=pl.BlockSpec((R, D), lambda i: (i, 0)),
        out_shape=jax.ShapeDtypeStruct((NP, D), F32),
    )(g2, s2, degp, b2r, clsp)


def kernel(x, x_h, x_s, edge_index, cls_embeddings, W1, b1, W2, b2):
    src = edge_index[0]
    dst = edge_index[1]
    pad = EPAD - E
    srcp = jnp.concatenate([src, jnp.full((pad,), N, jnp.int32)])
    dstp = jnp.concatenate([dst, N + (jnp.arange(pad, dtype=jnp.int32) % 96)])
    src2d = srcp.reshape(ECH, CB)
    dst2d = dstp.reshape(ECH, CB)
    src_l1 = jnp.stack([src2d, src2d + NP])
    zeros128 = jnp.zeros((NP, D), F32)
    b1r = b1.reshape(1, HID)
    b2r = b2.reshape(1, D)
    clsp = jnp.zeros((D, D), F32).at[0:NCLS].set(cls_embeddings)

    ones_blk = jnp.ones((CB, D), F32)
    degp = _make_count()(dst2d, zeros128, ones_blk)
    g1 = _mm1(x, x_h, x_s, W1, degp)
    s1 = _make_scatter(False)(g1.reshape(NSC * NP, D), src_l1, dst2d, zeros128)
    g2 = _epi1(g1, s1, degp, b1r, W2)
    s2 = _make_scatter(True)(g2, src2d, dst2d, zeros128)
    out = _epi2(g2, s2, degp, b2r, clsp)
    return out[0:N, 0:NCLS]
